# Initial kernel scaffold; baseline (speedup 1.0000x reference)
import functools

import jax
import jax.numpy as jnp
from jax import lax
from jax.experimental import pallas as pl
from jax.experimental.pallas import tpu as pltpu
from jax.experimental.pallas import tpu_sc as plsc

N = 10000
E = 160000
D_IN = 256
D_OUT = 256
D_EDGE = 16
HEADS = 8
HP = 16
SLOTS = 16
CH = 128
NCHUNK = E // CH
f32 = jnp.float32
i32 = jnp.int32

_mesh = plsc.VectorSubcoreMesh(core_axis_name="c", subcore_axis_name="s")


_BN1 = 500


def _proj_body(x_ref, w_ref, al_ref, ar_ref, ft_ref, el_ref, er_ref):
    y = jnp.dot(x_ref[...], w_ref[...], preferred_element_type=f32)
    for s in range(SLOTS):
        ft_ref[s] = y[:, s * 128:(s + 1) * 128]
    el_ref[...] = jnp.dot(y, al_ref[...], preferred_element_type=f32)
    er_ref[...] = jnp.dot(y, ar_ref[...], preferred_element_type=f32)


def _proj(nfeats, W_fc, AL16, AR16):
    return pl.pallas_call(
        _proj_body,
        grid=(N // _BN1,),
        in_specs=[
            pl.BlockSpec((_BN1, D_IN), lambda i: (i, 0)),
            pl.BlockSpec((D_IN, HEADS * D_OUT), lambda i: (0, 0)),
            pl.BlockSpec((HEADS * D_OUT, HP), lambda i: (0, 0)),
            pl.BlockSpec((HEADS * D_OUT, HP), lambda i: (0, 0)),
        ],
        out_specs=[
            pl.BlockSpec((SLOTS, _BN1, 128), lambda i: (0, i, 0)),
            pl.BlockSpec((_BN1, HP), lambda i: (i, 0)),
            pl.BlockSpec((_BN1, HP), lambda i: (i, 0)),
        ],
        out_shape=[
            jax.ShapeDtypeStruct((SLOTS, N, 128), f32),
            jax.ShapeDtypeStruct((N, HP), f32),
            jax.ShapeDtypeStruct((N, HP), f32),
        ],
    )(nfeats, W_fc, AL16, AR16)


_BE2 = 10000


def _ee_body(ef_ref, wfe_ref, ae_ref, hsel_ref, out_ref):
    ve = jnp.dot(wfe_ref[...] * ae_ref[...], hsel_ref[...],
                 preferred_element_type=f32)
    out_ref[...] = jnp.dot(ef_ref[...], ve, preferred_element_type=f32)


def _ee(efeats, W_fc_edge, ae_flat, HSEL):
    return pl.pallas_call(
        _ee_body,
        grid=(E // _BE2,),
        in_specs=[
            pl.BlockSpec((_BE2, D_EDGE), lambda i: (i, 0)),
            pl.BlockSpec((D_EDGE, HEADS * D_OUT), lambda i: (0, 0)),
            pl.BlockSpec((1, HEADS * D_OUT), lambda i: (0, 0)),
            pl.BlockSpec((HEADS * D_OUT, HP), lambda i: (0, 0)),
        ],
        out_specs=pl.BlockSpec((_BE2, HP), lambda i: (i, 0)),
        out_shape=jax.ShapeDtypeStruct((E, HP), f32),
    )(efeats, W_fc_edge, ae_flat, HSEL)


@functools.partial(
    pl.kernel,
    mesh=_mesh,
    out_type=[
        jax.ShapeDtypeStruct((HEADS, E), f32),
        jax.ShapeDtypeStruct((2, N, HP), f32),
    ],
    scratch_types=[
        pltpu.VMEM((CH,), i32),
        pltpu.VMEM((CH,), i32),
        pltpu.VMEM((CH, HP), f32),
        pltpu.VMEM((CH, HP), f32),
        pltpu.VMEM((CH, HP), f32),
        pltpu.VMEM((CH, HP), f32),
        pltpu.VMEM((HEADS, CH), f32),
        pltpu.VMEM((N // 16, HP), f32),
        pltpu.VMEM_SHARED((N, HP), f32),
        pltpu.SemaphoreType.DMA,
        pltpu.SemaphoreType.DMA,
    ],
)
def _edge_kernel(el_hbm, er_hbm, ee_hbm, src_hbm, dst_hbm, ext_out, dn_out,
                 srcv, dstv, elv, erv, eev, exv, extv, zbuf, dn_acc,
                 sem1, sem2):
    cid = lax.axis_index("c")
    sid = lax.axis_index("s")
    wid = cid * 16 + sid
    rows_per_tile = N // 16

    def _zb(i, carry):
        zbuf[i] = jnp.zeros((16,), f32)
        return carry

    lax.fori_loop(0, rows_per_tile, _zb, 0)
    pltpu.sync_copy(zbuf, dn_acc.at[pl.ds(sid * rows_per_tile, rows_per_tile)])
    plsc.subcore_barrier()

    nch = 39 + jnp.where(wid < NCHUNK - 39 * 32, 1, 0)

    def _chunk(i, carry):
        base = (wid + i * 32) * CH
        pltpu.sync_copy(src_hbm.at[pl.ds(base, CH)], srcv)
        pltpu.sync_copy(dst_hbm.at[pl.ds(base, CH)], dstv)
        pltpu.sync_copy(ee_hbm.at[pl.ds(base, CH), :], eev)
        d1 = pltpu.async_copy(el_hbm.at[srcv], elv, sem1)
        d2 = pltpu.async_copy(er_hbm.at[dstv], erv, sem2)
        d1.wait()
        d2.wait()

        def _row(j, c):
            x = elv[j] + erv[j] + eev[j]
            x = jnp.maximum(x, x * 0.2)
            exv[j] = jnp.exp(x)
            return c

        lax.fori_loop(0, CH, _row, 0)
        for h in range(HEADS):
            hc = jnp.full((16,), h, i32)
            for g in range(CH // 16):
                ridx = g * 16 + lax.iota(i32, 16)
                extv[h, pl.ds(g * 16, 16)] = plsc.load_gather(exv, [ridx, hc])
        pltpu.sync_copy(extv, ext_out.at[:, pl.ds(base, CH)])
        pltpu.sync_copy(exv, dn_acc.at[dstv], add=True)
        return carry

    lax.fori_loop(0, nch, _chunk, 0)
    plsc.subcore_barrier()
    pltpu.sync_copy(dn_acc.at[pl.ds(sid * rows_per_tile, rows_per_tile)],
                    dn_out.at[cid, pl.ds(sid * rows_per_tile, rows_per_tile)])


@functools.partial(
    pl.kernel,
    mesh=_mesh,
    out_type=jax.ShapeDtypeStruct((SLOTS, N, 128), f32),
    scratch_types=[
        pltpu.VMEM((CH,), i32),
        pltpu.VMEM((CH,), i32),
        pltpu.VMEM((CH,), i32),
        pltpu.VMEM((CH,), f32),
        pltpu.VMEM((CH, 128), f32),
        pltpu.VMEM((125, 128), f32),
        pltpu.VMEM_SHARED((N, 128), f32),
        pltpu.SemaphoreType.DMA,
    ],
)
def _msg_kernel(ft_hbm, ext_hbm, src_hbm, dst_hbm, rst_out,
                srcv, dstv, idxv, wv, rows, zbuf, acc, sem):
    cid = lax.axis_index("c")
    sid = lax.axis_index("s")
    rows_per_tile = N // 16

    def _zb(i, carry):
        for k in range(8):
            zbuf[i, pl.ds(k * 16, 16)] = jnp.zeros((16,), f32)
        return carry

    lax.fori_loop(0, 125, _zb, 0)

    nbase = NCHUNK // 16
    nch = nbase + jnp.where(sid < NCHUNK - nbase * 16, 1, 0)

    def _head(h, hcarry):
        slot = h * 2 + cid
        for q in range(rows_per_tile // 125):
            pltpu.sync_copy(
                zbuf, acc.at[pl.ds(sid * rows_per_tile + q * 125, 125)])
        plsc.subcore_barrier()

        def _chunk(i, carry):
            base = (sid + i * 16) * CH
            pltpu.sync_copy(src_hbm.at[pl.ds(base, CH)], srcv)
            pltpu.sync_copy(dst_hbm.at[pl.ds(base, CH)], dstv)
            pltpu.sync_copy(ext_hbm.at[h, pl.ds(base, CH)], wv)

            def _mkidx(g, c):
                idxv[pl.ds(g * 16, 16)] = srcv[pl.ds(g * 16, 16)] + slot * N
                return c

            lax.fori_loop(0, CH // 16, _mkidx, 0)
            pltpu.async_copy(ft_hbm.at[idxv], rows, sem).wait()

            def _row(j, c):
                ws = plsc.load_gather(wv, [jnp.full((16,), j, i32)])
                for k in range(8):
                    rows[j, pl.ds(k * 16, 16)] = rows[j, pl.ds(k * 16, 16)] * ws
                return c

            lax.fori_loop(0, CH, _row, 0)
            pltpu.sync_copy(rows, acc.at[dstv], add=True)
            return carry

        lax.fori_loop(0, nch, _chunk, 0)
        plsc.subcore_barrier()
        pltpu.sync_copy(acc.at[pl.ds(sid * rows_per_tile, rows_per_tile)],
                        rst_out.at[slot, pl.ds(sid * rows_per_tile,
                                               rows_per_tile)])
        plsc.subcore_barrier()
        return hcarry

    lax.fori_loop(0, HEADS, _head, 0)


_BN5 = 500


def _head_body(rst_ref, dn_ref, bias_ref, wps_ref, wpd_ref, s1_ref, s2_ref):
    dn = dn_ref[0] + dn_ref[1]
    dn = jnp.where(dn == 0.0, 1.0, dn)
    h0 = jnp.zeros((_BN5, 128), f32)
    h1 = jnp.zeros((_BN5, 128), f32)
    for s in range(SLOTS):
        hh, cc = s // 2, s % 2
        t = rst_ref[s] / dn[:, hh:hh + 1] + bias_ref[s][None, :]
        t = jnp.maximum(t, 0.0)
        if cc == 0:
            h0 = h0 + t
        else:
            h1 = h1 + t
    h0 = h0 * 0.125
    h1 = h1 * 0.125
    s1_ref[...] = (jnp.dot(h0, wps_ref[0], preferred_element_type=f32)
                   + jnp.dot(h1, wps_ref[1], preferred_element_type=f32))
    s2_ref[...] = (jnp.dot(h0, wpd_ref[0], preferred_element_type=f32)
                   + jnp.dot(h1, wpd_ref[1], preferred_element_type=f32))


def _head_mean(rst, dn, bias16, WpS, WpD):
    return pl.pallas_call(
        _head_body,
        grid=(N // _BN5,),
        in_specs=[
            pl.BlockSpec((SLOTS, _BN5, 128), lambda i: (0, i, 0)),
            pl.BlockSpec((2, _BN5, HP), lambda i: (0, i, 0)),
            pl.BlockSpec((SLOTS, 128), lambda i: (0, 0)),
            pl.BlockSpec((2, 128, HP), lambda i: (0, 0, 0)),
            pl.BlockSpec((2, 128, HP), lambda i: (0, 0, 0)),
        ],
        out_specs=[
            pl.BlockSpec((_BN5, HP), lambda i: (i, 0)),
            pl.BlockSpec((_BN5, HP), lambda i: (i, 0)),
        ],
        out_shape=[
            jax.ShapeDtypeStruct((N, HP), f32),
            jax.ShapeDtypeStruct((N, HP), f32),
        ],
    )(rst, dn, bias16, WpS, WpD)


@functools.partial(
    pl.kernel,
    mesh=_mesh,
    out_type=jax.ShapeDtypeStruct((E, HP), f32),
    scratch_types=[
        pltpu.VMEM((CH,), i32),
        pltpu.VMEM((CH,), i32),
        pltpu.VMEM((CH, HP), f32),
        pltpu.VMEM((CH, HP), f32),
        pltpu.VMEM((HP,), f32),
        pltpu.SemaphoreType.DMA,
        pltpu.SemaphoreType.DMA,
    ],
)
def _score_kernel(s1_hbm, s2_hbm, bp_hbm, src_hbm, dst_hbm, out_hbm,
                  srcv, dstv, av, bv, bpv, sem1, sem2):
    cid = lax.axis_index("c")
    sid = lax.axis_index("s")
    wid = cid * 16 + sid
    pltpu.sync_copy(bp_hbm, bpv)
    nch = 39 + jnp.where(wid < NCHUNK - 39 * 32, 1, 0)

    def _chunk(i, carry):
        base = (wid + i * 32) * CH
        pltpu.sync_copy(src_hbm.at[pl.ds(base, CH)], srcv)
        pltpu.sync_copy(dst_hbm.at[pl.ds(base, CH)], dstv)
        d1 = pltpu.async_copy(s1_hbm.at[srcv], av, sem1)
        d2 = pltpu.async_copy(s2_hbm.at[dstv], bv, sem2)
        d1.wait()
        d2.wait()
        bb = bpv[...]

        def _row(j, c):
            av[j] = av[j] + bv[j] + bb
            return c

        lax.fori_loop(0, CH, _row, 0)
        pltpu.sync_copy(av, out_hbm.at[pl.ds(base, CH), :])
        return carry

    lax.fori_loop(0, nch, _chunk, 0)


def kernel(nfeats, efeats, W_fc, attn_l, attn_r, W_fc_edge, attn_e, bias,
           W_pred, b_pred, edge_index):
    src = edge_index[0]
    dst = edge_index[1]

    col = lax.broadcasted_iota(i32, (HEADS * D_OUT, HP), 1)
    row_head = lax.broadcasted_iota(i32, (HEADS * D_OUT, HP), 0) // D_OUT
    sel = col == row_head
    AL16 = jnp.where(sel, attn_l.reshape(-1)[:, None], 0.0).astype(f32)
    AR16 = jnp.where(sel, attn_r.reshape(-1)[:, None], 0.0).astype(f32)
    HSEL = jnp.where(sel, 1.0, 0.0).astype(f32)
    bias16 = bias.reshape(SLOTS, 128)
    WpS = jnp.zeros((2, 128, HP), f32).at[:, :, :2].set(
        W_pred[:D_OUT].reshape(2, 128, 2))
    WpD = jnp.zeros((2, 128, HP), f32).at[:, :, :2].set(
        W_pred[D_OUT:].reshape(2, 128, 2))
    bp16 = jnp.zeros((HP,), f32).at[:2].set(b_pred)

    ft, el16, er16 = _proj(nfeats, W_fc, AL16, AR16)
    ee = _ee(efeats, W_fc_edge, attn_e.reshape(1, -1), HSEL)
    ext, dn = _edge_kernel(el16, er16, ee, src, dst)
    rst = _msg_kernel(ft.reshape(SLOTS * N, 128), ext, src, dst)
    s1, s2 = _head_mean(rst, dn, bias16, WpS, WpD)
    out16 = _score_kernel(s1, s2, bp16, src, dst)
    return out16[:, :2]


# speedup vs baseline: 6.8652x; 6.8652x over previous
import functools

import jax
import jax.numpy as jnp
from jax import lax
from jax.experimental import pallas as pl
from jax.experimental.pallas import tpu as pltpu
from jax.experimental.pallas import tpu_sc as plsc

N = 10000
E = 160000
D_IN = 256
D_OUT = 256
D_EDGE = 16
HEADS = 8
HP = 16
W = 128
SLOTS = 16
CH = 128
NCHUNK = E // CH
NP = 10240
RPT = NP // 16
f32 = jnp.float32
i32 = jnp.int32

_mesh = plsc.VectorSubcoreMesh(core_axis_name="c", subcore_axis_name="s")


_BN1 = 1000


def _proj_body(x_ref, w_ref, al_ref, ar_ref, ft_ref, el_ref, er_ref):
    y = jnp.dot(x_ref[...], w_ref[...], preferred_element_type=f32)
    for s in range(SLOTS):
        ft_ref[s] = y[:, s * 128:(s + 1) * 128]
    el_ref[...] = jnp.dot(y, al_ref[...], preferred_element_type=f32)
    er_ref[...] = jnp.dot(y, ar_ref[...], preferred_element_type=f32)


def _proj(nfeats, W_fc, AL, AR):
    return pl.pallas_call(
        _proj_body,
        grid=(N // _BN1,),
        in_specs=[
            pl.BlockSpec((_BN1, D_IN), lambda i: (i, 0)),
            pl.BlockSpec((D_IN, HEADS * D_OUT), lambda i: (0, 0)),
            pl.BlockSpec((HEADS * D_OUT, W), lambda i: (0, 0)),
            pl.BlockSpec((HEADS * D_OUT, W), lambda i: (0, 0)),
        ],
        out_specs=[
            pl.BlockSpec((SLOTS, _BN1, 128), lambda i: (0, i, 0)),
            pl.BlockSpec((_BN1, W), lambda i: (i, 0)),
            pl.BlockSpec((_BN1, W), lambda i: (i, 0)),
        ],
        out_shape=[
            jax.ShapeDtypeStruct((SLOTS, N, 128), f32),
            jax.ShapeDtypeStruct((N, W), f32),
            jax.ShapeDtypeStruct((N, W), f32),
        ],
    )(nfeats, W_fc, AL, AR)


_BE2 = 10000


def _ee_body(ef_ref, wfe_ref, ae_ref, hsel_ref, out_ref):
    ve = jnp.dot(wfe_ref[...] * ae_ref[...], hsel_ref[...],
                 preferred_element_type=f32)
    out_ref[...] = jnp.dot(ef_ref[...], ve, preferred_element_type=f32)


def _ee(efeats, W_fc_edge, ae_flat, HSEL):
    return pl.pallas_call(
        _ee_body,
        grid=(E // _BE2,),
        in_specs=[
            pl.BlockSpec((_BE2, D_EDGE), lambda i: (i, 0)),
            pl.BlockSpec((D_EDGE, HEADS * D_OUT), lambda i: (0, 0)),
            pl.BlockSpec((1, HEADS * D_OUT), lambda i: (0, 0)),
            pl.BlockSpec((HEADS * D_OUT, HP), lambda i: (0, 0)),
        ],
        out_specs=pl.BlockSpec((_BE2, HP), lambda i: (i, 0)),
        out_shape=jax.ShapeDtypeStruct((E, HP), f32),
    )(efeats, W_fc_edge, ae_flat, HSEL)


@functools.partial(
    pl.kernel,
    mesh=_mesh,
    out_type=jax.ShapeDtypeStruct((E, HP), f32),
    scratch_types=[
        pltpu.VMEM((CH,), i32),
        pltpu.VMEM((CH,), i32),
        pltpu.VMEM((CH, W), f32),
        pltpu.VMEM((CH, W), f32),
        pltpu.VMEM((CH, HP), f32),
        pltpu.VMEM((CH, HP), f32),
        pltpu.SemaphoreType.DMA,
        pltpu.SemaphoreType.DMA,
    ],
)
def _edge_kernel(el_hbm, er_hbm, ee_hbm, src_hbm, dst_hbm, exr_out,
                 srcv, dstv, elv, erv, eev, exv16, sem1, sem2):
    cid = lax.axis_index("c")
    sid = lax.axis_index("s")
    wid = cid * 16 + sid
    nch = 39 + jnp.where(wid < NCHUNK - 39 * 32, 1, 0)

    def _chunk(i, carry):
        base = (wid + i * 32) * CH
        pltpu.sync_copy(src_hbm.at[pl.ds(base, CH)], srcv)
        pltpu.sync_copy(dst_hbm.at[pl.ds(base, CH)], dstv)
        pltpu.sync_copy(ee_hbm.at[pl.ds(base, CH), :], eev)
        d1 = pltpu.async_copy(el_hbm.at[srcv], elv, sem1)
        d2 = pltpu.async_copy(er_hbm.at[dstv], erv, sem2)
        d1.wait()
        d2.wait()

        def _row(j, c):
            x = elv[j, pl.ds(0, 16)] + erv[j, pl.ds(0, 16)] + eev[j]
            x = jnp.maximum(x, x * 0.2)
            exv16[j] = jnp.exp(x)
            return c

        lax.fori_loop(0, CH, _row, 0)
        pltpu.sync_copy(exv16, exr_out.at[pl.ds(base, CH), :])
        return carry

    lax.fori_loop(0, nch, _chunk, 0)


_BT = 16000


def _tr_body(x_ref, o_ref):
    o_ref[...] = x_ref[...].T[:HEADS, :]


def _transpose_ex(ex_rows):
    return pl.pallas_call(
        _tr_body,
        grid=(E // _BT,),
        in_specs=[pl.BlockSpec((_BT, HP), lambda i: (i, 0))],
        out_specs=pl.BlockSpec((HEADS, _BT), lambda i: (0, i)),
        out_shape=jax.ShapeDtypeStruct((HEADS, E), f32),
    )(ex_rows)


@functools.partial(
    pl.kernel,
    mesh=_mesh,
    out_type=[
        jax.ShapeDtypeStruct((SLOTS, NP, 128), f32),
        jax.ShapeDtypeStruct((2, NP, W), f32),
    ],
    scratch_types=[
        pltpu.VMEM((CH,), i32),
        pltpu.VMEM((CH,), i32),
        pltpu.VMEM((CH,), i32),
        pltpu.VMEM((CH,), f32),
        pltpu.VMEM((CH, HP), f32),
        pltpu.VMEM((CH, 128), f32),
        pltpu.VMEM((64, 128), f32),
        pltpu.VMEM_SHARED((NP, 128), f32),
        pltpu.SemaphoreType.DMA,
    ],
)
def _msg_kernel(ft_hbm, ext_hbm, exr_hbm, src_hbm, dst_hbm, rst_out, dn_out,
                srcv, dstv, idxv, wv, exv16, rows, zbuf, acc, sem):
    cid = lax.axis_index("c")
    sid = lax.axis_index("s")

    def _zb(i, carry):
        for k in range(8):
            zbuf[i, pl.ds(k * 16, 16)] = jnp.zeros((16,), f32)
        return carry

    lax.fori_loop(0, 64, _zb, 0)

    nbase = NCHUNK // 16
    nch = nbase + jnp.where(sid < NCHUNK - nbase * 16, 1, 0)

    def _zero_acc():
        for q in range(RPT // 64):
            pltpu.sync_copy(
                zbuf, acc.at[pl.ds(sid * RPT + q * 64, 64)])
        plsc.subcore_barrier()

    _zero_acc()

    def _dchunk(i, carry):
        base = (sid + i * 16) * CH
        pltpu.sync_copy(dst_hbm.at[pl.ds(base, CH)], dstv)
        pltpu.sync_copy(exr_hbm.at[pl.ds(base, CH), :], exv16)

        def _drow(j, c):
            v = exv16[j]
            for k in range(8):
                rows[j, pl.ds(k * 16, 16)] = v
            return c

        lax.fori_loop(0, CH, _drow, 0)
        pltpu.sync_copy(rows, acc.at[dstv], add=True)
        return carry

    lax.fori_loop(0, nch, _dchunk, 0)
    plsc.subcore_barrier()
    pltpu.sync_copy(acc.at[pl.ds(sid * RPT, RPT)],
                    dn_out.at[cid, pl.ds(sid * RPT, RPT)])
    plsc.subcore_barrier()

    def _head(h, hcarry):
        slot = h * 2 + cid
        _zero_acc()

        def _chunk(i, carry):
            base = (sid + i * 16) * CH
            pltpu.sync_copy(src_hbm.at[pl.ds(base, CH)], srcv)
            pltpu.sync_copy(dst_hbm.at[pl.ds(base, CH)], dstv)
            pltpu.sync_copy(ext_hbm.at[h, pl.ds(base, CH)], wv)

            def _mkidx(g, c):
                idxv[pl.ds(g * 16, 16)] = srcv[pl.ds(g * 16, 16)] + slot * N
                return c

            lax.fori_loop(0, CH // 16, _mkidx, 0)
            pltpu.async_copy(ft_hbm.at[idxv], rows, sem).wait()

            def _grp(g, c):
                wg = wv[pl.ds(g * 16, 16)]
                for l in range(16):
                    ws = wg[l]
                    j = g * 16 + l
                    for k in range(8):
                        rows[j, pl.ds(k * 16, 16)] = (
                            rows[j, pl.ds(k * 16, 16)] * ws)
                return c

            lax.fori_loop(0, CH // 16, _grp, 0)
            pltpu.sync_copy(rows, acc.at[dstv], add=True)
            return carry

        lax.fori_loop(0, nch, _chunk, 0)
        plsc.subcore_barrier()
        pltpu.sync_copy(acc.at[pl.ds(sid * RPT, RPT)],
                        rst_out.at[slot, pl.ds(sid * RPT, RPT)])
        plsc.subcore_barrier()
        return hcarry

    lax.fori_loop(0, HEADS, _head, 0)


_BN5 = 1024


def _head_body(rst_ref, dn_ref, bias_ref, wps_ref, wpd_ref, s1_ref, s2_ref):
    dn = (dn_ref[0] + dn_ref[1]) * 0.5
    dn = jnp.where(dn == 0.0, 1.0, dn)
    h0 = jnp.zeros((_BN5, 128), f32)
    h1 = jnp.zeros((_BN5, 128), f32)
    for s in range(SLOTS):
        hh, cc = s // 2, s % 2
        t = rst_ref[s] / dn[:, hh:hh + 1] + bias_ref[s][None, :]
        t = jnp.maximum(t, 0.0)
        if cc == 0:
            h0 = h0 + t
        else:
            h1 = h1 + t
    h0 = h0 * 0.125
    h1 = h1 * 0.125
    s1_ref[...] = (jnp.dot(h0, wps_ref[0], preferred_element_type=f32)
                   + jnp.dot(h1, wps_ref[1], preferred_element_type=f32))
    s2_ref[...] = (jnp.dot(h0, wpd_ref[0], preferred_element_type=f32)
                   + jnp.dot(h1, wpd_ref[1], preferred_element_type=f32))


def _head_mean(rst, dn, bias16, WpS, WpD):
    return pl.pallas_call(
        _head_body,
        grid=(NP // _BN5,),
        in_specs=[
            pl.BlockSpec((SLOTS, _BN5, 128), lambda i: (0, i, 0)),
            pl.BlockSpec((2, _BN5, W), lambda i: (0, i, 0)),
            pl.BlockSpec((SLOTS, 128), lambda i: (0, 0)),
            pl.BlockSpec((2, 128, W), lambda i: (0, 0, 0)),
            pl.BlockSpec((2, 128, W), lambda i: (0, 0, 0)),
        ],
        out_specs=[
            pl.BlockSpec((_BN5, W), lambda i: (i, 0)),
            pl.BlockSpec((_BN5, W), lambda i: (i, 0)),
        ],
        out_shape=[
            jax.ShapeDtypeStruct((NP, W), f32),
            jax.ShapeDtypeStruct((NP, W), f32),
        ],
    )(rst, dn, bias16, WpS, WpD)


@functools.partial(
    pl.kernel,
    mesh=_mesh,
    out_type=jax.ShapeDtypeStruct((E, HP), f32),
    scratch_types=[
        pltpu.VMEM((CH,), i32),
        pltpu.VMEM((CH,), i32),
        pltpu.VMEM((CH, W), f32),
        pltpu.VMEM((CH, W), f32),
        pltpu.VMEM((CH, HP), f32),
        pltpu.VMEM((W,), f32),
        pltpu.SemaphoreType.DMA,
        pltpu.SemaphoreType.DMA,
    ],
)
def _score_kernel(s1_hbm, s2_hbm, bp_hbm, src_hbm, dst_hbm, out_hbm,
                  srcv, dstv, av, bv, ov, bpv, sem1, sem2):
    cid = lax.axis_index("c")
    sid = lax.axis_index("s")
    wid = cid * 16 + sid
    pltpu.sync_copy(bp_hbm, bpv)
    nch = 39 + jnp.where(wid < NCHUNK - 39 * 32, 1, 0)

    def _chunk(i, carry):
        base = (wid + i * 32) * CH
        pltpu.sync_copy(src_hbm.at[pl.ds(base, CH)], srcv)
        pltpu.sync_copy(dst_hbm.at[pl.ds(base, CH)], dstv)
        d1 = pltpu.async_copy(s1_hbm.at[srcv], av, sem1)
        d2 = pltpu.async_copy(s2_hbm.at[dstv], bv, sem2)
        d1.wait()
        d2.wait()
        bb = bpv[pl.ds(0, 16)]

        def _row(j, c):
            ov[j] = av[j, pl.ds(0, 16)] + bv[j, pl.ds(0, 16)] + bb
            return c

        lax.fori_loop(0, CH, _row, 0)
        pltpu.sync_copy(ov, out_hbm.at[pl.ds(base, CH), :])
        return carry

    lax.fori_loop(0, nch, _chunk, 0)


def kernel(nfeats, efeats, W_fc, attn_l, attn_r, W_fc_edge, attn_e, bias,
           W_pred, b_pred, edge_index):
    src = edge_index[0]
    dst = edge_index[1]

    colw = lax.broadcasted_iota(i32, (HEADS * D_OUT, W), 1)
    roww = lax.broadcasted_iota(i32, (HEADS * D_OUT, W), 0) // D_OUT
    selw = colw == roww
    AL = jnp.where(selw, attn_l.reshape(-1)[:, None], 0.0).astype(f32)
    AR = jnp.where(selw, attn_r.reshape(-1)[:, None], 0.0).astype(f32)
    col16 = lax.broadcasted_iota(i32, (HEADS * D_OUT, HP), 1)
    row16 = lax.broadcasted_iota(i32, (HEADS * D_OUT, HP), 0) // D_OUT
    HSEL = jnp.where(col16 == row16, 1.0, 0.0).astype(f32)
    bias16 = bias.reshape(SLOTS, 128)
    WpS = jnp.zeros((2, 128, W), f32).at[:, :, :2].set(
        W_pred[:D_OUT].reshape(2, 128, 2))
    WpD = jnp.zeros((2, 128, W), f32).at[:, :, :2].set(
        W_pred[D_OUT:].reshape(2, 128, 2))
    bp = jnp.zeros((W,), f32).at[:2].set(b_pred)

    ft, el, er = _proj(nfeats, W_fc, AL, AR)
    ee = _ee(efeats, W_fc_edge, attn_e.reshape(1, -1), HSEL)
    ex_rows = _edge_kernel(el, er, ee, src, dst)
    ext = _transpose_ex(ex_rows)
    rst, dn = _msg_kernel(ft.reshape(SLOTS * N, 128), ext, ex_rows, src, dst)
    s1, s2 = _head_mean(rst, dn, bias16, WpS, WpD)
    out16 = _score_kernel(s1, s2, bp, src, dst)
    return out16[:, :2]


# speedup vs baseline: 10.1935x; 1.4848x over previous
import functools

import jax
import jax.numpy as jnp
from jax import lax
from jax.experimental import pallas as pl
from jax.experimental.pallas import tpu as pltpu
from jax.experimental.pallas import tpu_sc as plsc

N = 10000
E = 160000
D_IN = 256
D_OUT = 256
D_EDGE = 16
HEADS = 8
HP = 16
W = 128
SLOTS = 16
CH = 128
NCHUNK = E // CH
NP = 10240
RPT = NP // 16
f32 = jnp.float32
i32 = jnp.int32

_mesh = plsc.VectorSubcoreMesh(core_axis_name="c", subcore_axis_name="s")


_BN1 = 1000


def _proj_body(x_ref, w_ref, al_ref, ar_ref, ft_ref, el_ref, er_ref):
    y = jnp.dot(x_ref[...], w_ref[...], preferred_element_type=f32)
    for s in range(SLOTS):
        ft_ref[s] = y[:, s * 128:(s + 1) * 128]
    el_ref[...] = jnp.dot(y, al_ref[...], preferred_element_type=f32)
    er_ref[...] = jnp.dot(y, ar_ref[...], preferred_element_type=f32)


def _proj(nfeats, W_fc, AL, AR):
    return pl.pallas_call(
        _proj_body,
        grid=(N // _BN1,),
        in_specs=[
            pl.BlockSpec((_BN1, D_IN), lambda i: (i, 0)),
            pl.BlockSpec((D_IN, HEADS * D_OUT), lambda i: (0, 0)),
            pl.BlockSpec((HEADS * D_OUT, W), lambda i: (0, 0)),
            pl.BlockSpec((HEADS * D_OUT, W), lambda i: (0, 0)),
        ],
        out_specs=[
            pl.BlockSpec((SLOTS, _BN1, 128), lambda i: (0, i, 0)),
            pl.BlockSpec((_BN1, W), lambda i: (i, 0)),
            pl.BlockSpec((_BN1, W), lambda i: (i, 0)),
        ],
        out_shape=[
            jax.ShapeDtypeStruct((SLOTS, N, 128), f32),
            jax.ShapeDtypeStruct((N, W), f32),
            jax.ShapeDtypeStruct((N, W), f32),
        ],
    )(nfeats, W_fc, AL, AR)


_BE2 = 10000


def _ee_body(ef_ref, wfe_ref, ae_ref, hsel_ref, out_ref):
    ve = jnp.dot(wfe_ref[...] * ae_ref[...], hsel_ref[...],
                 preferred_element_type=f32)
    out_ref[...] = jnp.dot(ef_ref[...], ve, preferred_element_type=f32)


def _ee(efeats, W_fc_edge, ae_flat, HSEL):
    return pl.pallas_call(
        _ee_body,
        grid=(E // _BE2,),
        in_specs=[
            pl.BlockSpec((_BE2, D_EDGE), lambda i: (i, 0)),
            pl.BlockSpec((D_EDGE, HEADS * D_OUT), lambda i: (0, 0)),
            pl.BlockSpec((1, HEADS * D_OUT), lambda i: (0, 0)),
            pl.BlockSpec((HEADS * D_OUT, HP), lambda i: (0, 0)),
        ],
        out_specs=pl.BlockSpec((_BE2, HP), lambda i: (i, 0)),
        out_shape=jax.ShapeDtypeStruct((E, HP), f32),
    )(efeats, W_fc_edge, ae_flat, HSEL)


@functools.partial(
    pl.kernel,
    mesh=_mesh,
    out_type=jax.ShapeDtypeStruct((E, HP), f32),
    scratch_types=[
        pltpu.VMEM((CH,), i32),
        pltpu.VMEM((CH,), i32),
        pltpu.VMEM((CH, W), f32),
        pltpu.VMEM((CH, W), f32),
        pltpu.VMEM((CH, HP), f32),
        pltpu.VMEM((CH, HP), f32),
        pltpu.SemaphoreType.DMA,
        pltpu.SemaphoreType.DMA,
    ],
)
def _edge_kernel(el_hbm, er_hbm, ee_hbm, src_hbm, dst_hbm, exr_out,
                 srcv, dstv, elv, erv, eev, exv16, sem1, sem2):
    cid = lax.axis_index("c")
    sid = lax.axis_index("s")
    wid = cid * 16 + sid
    nch = 39 + jnp.where(wid < NCHUNK - 39 * 32, 1, 0)

    def _chunk(i, carry):
        base = (wid + i * 32) * CH
        pltpu.sync_copy(src_hbm.at[pl.ds(base, CH)], srcv)
        pltpu.sync_copy(dst_hbm.at[pl.ds(base, CH)], dstv)
        pltpu.sync_copy(ee_hbm.at[pl.ds(base, CH), :], eev)
        d1 = pltpu.async_copy(el_hbm.at[srcv], elv, sem1)
        d2 = pltpu.async_copy(er_hbm.at[dstv], erv, sem2)
        d1.wait()
        d2.wait()

        def _row(j, c):
            x = elv[j, pl.ds(0, 16)] + erv[j, pl.ds(0, 16)] + eev[j]
            x = jnp.maximum(x, x * 0.2)
            exv16[j] = jnp.exp(x)
            return c

        lax.fori_loop(0, CH, _row, 0)
        pltpu.sync_copy(exv16, exr_out.at[pl.ds(base, CH), :])
        return carry

    lax.fori_loop(0, nch, _chunk, 0)


_BT = 16000


def _tr_body(x_ref, o_ref):
    o_ref[...] = x_ref[...].T[:HEADS, :]


def _transpose_ex(ex_rows):
    return pl.pallas_call(
        _tr_body,
        grid=(E // _BT,),
        in_specs=[pl.BlockSpec((_BT, HP), lambda i: (i, 0))],
        out_specs=pl.BlockSpec((HEADS, _BT), lambda i: (0, i)),
        out_shape=jax.ShapeDtypeStruct((HEADS, E), f32),
    )(ex_rows)


_CH4 = 64
_PAIRS = 78


@functools.partial(
    pl.kernel,
    mesh=_mesh,
    out_type=[
        jax.ShapeDtypeStruct((SLOTS, NP, 128), f32),
        jax.ShapeDtypeStruct((2, NP, W), f32),
    ],
    scratch_types=[
        pltpu.VMEM((_CH4,), i32), pltpu.VMEM((_CH4,), i32),
        pltpu.VMEM((_CH4,), i32), pltpu.VMEM((_CH4,), f32),
        pltpu.VMEM((_CH4, 128), f32), pltpu.VMEM((_CH4, HP), f32),
        pltpu.VMEM((_CH4,), i32), pltpu.VMEM((_CH4,), i32),
        pltpu.VMEM((_CH4,), i32), pltpu.VMEM((_CH4,), f32),
        pltpu.VMEM((_CH4, 128), f32), pltpu.VMEM((_CH4, HP), f32),
        pltpu.VMEM((64, 128), f32),
        pltpu.VMEM_SHARED((NP, 128), f32),
        pltpu.SemaphoreType.DMA, pltpu.SemaphoreType.DMA,
        pltpu.SemaphoreType.DMA, pltpu.SemaphoreType.DMA,
        pltpu.SemaphoreType.DMA, pltpu.SemaphoreType.DMA,
    ],
)
def _msg_kernel(ft_hbm, ext_hbm, exr_hbm, src_hbm, dst_hbm, rst_out, dn_out,
                srcA, dstA, idxA, wA, rowsA, exA,
                srcB, dstB, idxB, wB, rowsB, exB,
                zbuf, acc, mA, gA, sA, mB, gB, sB):
    cid = lax.axis_index("c")
    sid = lax.axis_index("s")
    A = (srcA, dstA, idxA, wA, rowsA, exA, mA, gA, sA)
    B = (srcB, dstB, idxB, wB, rowsB, exB, mB, gB, sB)

    def _zb(i, carry):
        for k in range(8):
            zbuf[i, pl.ds(k * 16, 16)] = jnp.zeros((16,), f32)
        return carry

    lax.fori_loop(0, 64, _zb, 0)

    bc = 156 * sid + jnp.minimum(sid, 4)

    def _zero_acc():
        for q in range(RPT // 64):
            pltpu.sync_copy(
                zbuf, acc.at[pl.ds(sid * RPT + q * 64, 64)])
        plsc.subcore_barrier()

    def meta_issue(S, c, h):
        srcX, dstX, idxX, wX, rowsX, exX, mX, gX, sX = S
        base = c * _CH4
        pltpu.async_copy(src_hbm.at[pl.ds(base, _CH4)], srcX, mX)
        pltpu.async_copy(dst_hbm.at[pl.ds(base, _CH4)], dstX, mX)
        pltpu.async_copy(ext_hbm.at[h, pl.ds(base, _CH4)], wX, mX)

    def meta_wait(S):
        srcX, dstX, idxX, wX, rowsX, exX, mX, gX, sX = S
        pltpu.make_async_copy(src_hbm.at[pl.ds(0, _CH4)], srcX, mX).wait()
        pltpu.make_async_copy(dst_hbm.at[pl.ds(0, _CH4)], dstX, mX).wait()
        pltpu.make_async_copy(ext_hbm.at[0, pl.ds(0, _CH4)], wX, mX).wait()

    def gather_issue(S, slot):
        srcX, dstX, idxX, wX, rowsX, exX, mX, gX, sX = S

        def _mkidx(g, c):
            idxX[pl.ds(g * 16, 16)] = srcX[pl.ds(g * 16, 16)] + slot * N
            return c

        lax.fori_loop(0, _CH4 // 16, _mkidx, 0)
        pltpu.async_copy(ft_hbm.at[idxX], rowsX, gX)

    def gather_wait(S):
        srcX, dstX, idxX, wX, rowsX, exX, mX, gX, sX = S
        pltpu.make_async_copy(ft_hbm.at[idxX], rowsX, gX).wait()

    def mult(S):
        srcX, dstX, idxX, wX, rowsX, exX, mX, gX, sX = S

        def _grp(g, c):
            wg = wX[pl.ds(g * 16, 16)]
            for l in range(16):
                ws = wg[l]
                j = g * 16 + l
                for k in range(8):
                    rowsX[j, pl.ds(k * 16, 16)] = (
                        rowsX[j, pl.ds(k * 16, 16)] * ws)
            return c

        lax.fori_loop(0, _CH4 // 16, _grp, 0)

    def scatter_issue(S):
        srcX, dstX, idxX, wX, rowsX, exX, mX, gX, sX = S
        pltpu.async_copy(rowsX, acc.at[dstX], sX, add=True)

    def scatter_wait(S):
        srcX, dstX, idxX, wX, rowsX, exX, mX, gX, sX = S
        pltpu.make_async_copy(ft_hbm.at[idxX], rowsX, sX).wait()

    def dn_meta_issue(S, c):
        srcX, dstX, idxX, wX, rowsX, exX, mX, gX, sX = S
        base = c * _CH4
        pltpu.async_copy(dst_hbm.at[pl.ds(base, _CH4)], dstX, mX)
        pltpu.async_copy(exr_hbm.at[pl.ds(base, _CH4), :], exX, mX)

    def dn_meta_wait(S):
        srcX, dstX, idxX, wX, rowsX, exX, mX, gX, sX = S
        pltpu.make_async_copy(dst_hbm.at[pl.ds(0, _CH4)], dstX, mX).wait()
        pltpu.make_async_copy(exr_hbm.at[pl.ds(0, _CH4), :], exX, mX).wait()

    def dn_fill(S):
        srcX, dstX, idxX, wX, rowsX, exX, mX, gX, sX = S

        def _drow(j, c):
            v = exX[j]
            for k in range(8):
                rowsX[j, pl.ds(k * 16, 16)] = v
            return c

        lax.fori_loop(0, _CH4, _drow, 0)

    _zero_acc()
    dn_meta_issue(A, bc)
    dn_meta_issue(B, bc + 1)

    def _dnbody(p, carry):
        dn_meta_wait(A)
        dn_fill(A)
        scatter_issue(A)
        dn_meta_wait(B)
        dn_fill(B)
        scatter_issue(B)

        @pl.when(p < _PAIRS - 1)
        def _():
            scatter_wait(A)
            dn_meta_issue(A, bc + 2 * p + 2)
            scatter_wait(B)
            dn_meta_issue(B, bc + 2 * p + 3)

        @pl.when(p == _PAIRS - 1)
        def _():
            scatter_wait(A)
            scatter_wait(B)

        return carry

    lax.fori_loop(0, _PAIRS, _dnbody, 0)

    @pl.when(sid < 4)
    def _():
        dn_meta_issue(A, bc + 156)
        dn_meta_wait(A)
        dn_fill(A)
        scatter_issue(A)
        scatter_wait(A)

    plsc.subcore_barrier()
    pltpu.sync_copy(acc.at[pl.ds(sid * RPT, RPT)],
                    dn_out.at[cid, pl.ds(sid * RPT, RPT)])
    plsc.subcore_barrier()

    def _head(h, hcarry):
        slot = h * 2 + cid
        _zero_acc()
        meta_issue(A, bc, h)
        meta_issue(B, bc + 1, h)
        meta_wait(A)
        gather_issue(A, slot)

        def _body(p, carry):
            meta_wait(B)
            gather_issue(B, slot)
            gather_wait(A)
            mult(A)
            scatter_issue(A)
            gather_wait(B)
            mult(B)
            scatter_issue(B)

            @pl.when(p < _PAIRS - 1)
            def _():
                scatter_wait(A)
                meta_issue(A, bc + 2 * p + 2, h)
                meta_wait(A)
                gather_issue(A, slot)
                scatter_wait(B)
                meta_issue(B, bc + 2 * p + 3, h)

            @pl.when(p == _PAIRS - 1)
            def _():
                scatter_wait(A)
                scatter_wait(B)

            return carry

        lax.fori_loop(0, _PAIRS, _body, 0)

        @pl.when(sid < 4)
        def _():
            meta_issue(A, bc + 156, h)
            meta_wait(A)
            gather_issue(A, slot)
            gather_wait(A)
            mult(A)
            scatter_issue(A)
            scatter_wait(A)

        plsc.subcore_barrier()
        pltpu.sync_copy(acc.at[pl.ds(sid * RPT, RPT)],
                        rst_out.at[slot, pl.ds(sid * RPT, RPT)])
        plsc.subcore_barrier()
        return hcarry

    lax.fori_loop(0, HEADS, _head, 0)


_BN5 = 1024


def _head_body(rst_ref, dn_ref, bias_ref, wps_ref, wpd_ref, s1_ref, s2_ref):
    dn = (dn_ref[0] + dn_ref[1]) * 0.5
    dn = jnp.where(dn == 0.0, 1.0, dn)
    h0 = jnp.zeros((_BN5, 128), f32)
    h1 = jnp.zeros((_BN5, 128), f32)
    for s in range(SLOTS):
        hh, cc = s // 2, s % 2
        t = rst_ref[s] / dn[:, hh:hh + 1] + bias_ref[s][None, :]
        t = jnp.maximum(t, 0.0)
        if cc == 0:
            h0 = h0 + t
        else:
            h1 = h1 + t
    h0 = h0 * 0.125
    h1 = h1 * 0.125
    s1_ref[...] = (jnp.dot(h0, wps_ref[0], preferred_element_type=f32)
                   + jnp.dot(h1, wps_ref[1], preferred_element_type=f32))
    s2_ref[...] = (jnp.dot(h0, wpd_ref[0], preferred_element_type=f32)
                   + jnp.dot(h1, wpd_ref[1], preferred_element_type=f32))


def _head_mean(rst, dn, bias16, WpS, WpD):
    return pl.pallas_call(
        _head_body,
        grid=(NP // _BN5,),
        in_specs=[
            pl.BlockSpec((SLOTS, _BN5, 128), lambda i: (0, i, 0)),
            pl.BlockSpec((2, _BN5, W), lambda i: (0, i, 0)),
            pl.BlockSpec((SLOTS, 128), lambda i: (0, 0)),
            pl.BlockSpec((2, 128, W), lambda i: (0, 0, 0)),
            pl.BlockSpec((2, 128, W), lambda i: (0, 0, 0)),
        ],
        out_specs=[
            pl.BlockSpec((_BN5, W), lambda i: (i, 0)),
            pl.BlockSpec((_BN5, W), lambda i: (i, 0)),
        ],
        out_shape=[
            jax.ShapeDtypeStruct((NP, W), f32),
            jax.ShapeDtypeStruct((NP, W), f32),
        ],
    )(rst, dn, bias16, WpS, WpD)


@functools.partial(
    pl.kernel,
    mesh=_mesh,
    out_type=jax.ShapeDtypeStruct((E, HP), f32),
    scratch_types=[
        pltpu.VMEM((CH,), i32),
        pltpu.VMEM((CH,), i32),
        pltpu.VMEM((CH, W), f32),
        pltpu.VMEM((CH, W), f32),
        pltpu.VMEM((CH, HP), f32),
        pltpu.VMEM((W,), f32),
        pltpu.SemaphoreType.DMA,
        pltpu.SemaphoreType.DMA,
    ],
)
def _score_kernel(s1_hbm, s2_hbm, bp_hbm, src_hbm, dst_hbm, out_hbm,
                  srcv, dstv, av, bv, ov, bpv, sem1, sem2):
    cid = lax.axis_index("c")
    sid = lax.axis_index("s")
    wid = cid * 16 + sid
    pltpu.sync_copy(bp_hbm, bpv)
    nch = 39 + jnp.where(wid < NCHUNK - 39 * 32, 1, 0)

    def _chunk(i, carry):
        base = (wid + i * 32) * CH
        pltpu.sync_copy(src_hbm.at[pl.ds(base, CH)], srcv)
        pltpu.sync_copy(dst_hbm.at[pl.ds(base, CH)], dstv)
        d1 = pltpu.async_copy(s1_hbm.at[srcv], av, sem1)
        d2 = pltpu.async_copy(s2_hbm.at[dstv], bv, sem2)
        d1.wait()
        d2.wait()
        bb = bpv[pl.ds(0, 16)]

        def _row(j, c):
            ov[j] = av[j, pl.ds(0, 16)] + bv[j, pl.ds(0, 16)] + bb
            return c

        lax.fori_loop(0, CH, _row, 0)
        pltpu.sync_copy(ov, out_hbm.at[pl.ds(base, CH), :])
        return carry

    lax.fori_loop(0, nch, _chunk, 0)


def kernel(nfeats, efeats, W_fc, attn_l, attn_r, W_fc_edge, attn_e, bias,
           W_pred, b_pred, edge_index):
    src = edge_index[0]
    dst = edge_index[1]

    colw = lax.broadcasted_iota(i32, (HEADS * D_OUT, W), 1)
    roww = lax.broadcasted_iota(i32, (HEADS * D_OUT, W), 0) // D_OUT
    selw = colw == roww
    AL = jnp.where(selw, attn_l.reshape(-1)[:, None], 0.0).astype(f32)
    AR = jnp.where(selw, attn_r.reshape(-1)[:, None], 0.0).astype(f32)
    col16 = lax.broadcasted_iota(i32, (HEADS * D_OUT, HP), 1)
    row16 = lax.broadcasted_iota(i32, (HEADS * D_OUT, HP), 0) // D_OUT
    HSEL = jnp.where(col16 == row16, 1.0, 0.0).astype(f32)
    bias16 = bias.reshape(SLOTS, 128)
    WpS = jnp.zeros((2, 128, W), f32).at[:, :, :2].set(
        W_pred[:D_OUT].reshape(2, 128, 2))
    WpD = jnp.zeros((2, 128, W), f32).at[:, :, :2].set(
        W_pred[D_OUT:].reshape(2, 128, 2))
    bp = jnp.zeros((W,), f32).at[:2].set(b_pred)

    ft, el, er = _proj(nfeats, W_fc, AL, AR)
    ee = _ee(efeats, W_fc_edge, attn_e.reshape(1, -1), HSEL)
    ex_rows = _edge_kernel(el, er, ee, src, dst)
    ext = _transpose_ex(ex_rows)
    rst, dn = _msg_kernel(ft.reshape(SLOTS * N, 128), ext, ex_rows, src, dst)
    s1, s2 = _head_mean(rst, dn, bias16, WpS, WpD)
    out16 = _score_kernel(s1, s2, bp, src, dst)
    return out16[:, :2]


# speedup vs baseline: 10.8527x; 1.0647x over previous
import functools

import jax
import jax.numpy as jnp
from jax import lax
from jax.experimental import pallas as pl
from jax.experimental.pallas import tpu as pltpu
from jax.experimental.pallas import tpu_sc as plsc

N = 10000
E = 160000
D_IN = 256
D_OUT = 256
D_EDGE = 16
HEADS = 8
HP = 16
W = 128
SLOTS = 16
CH = 128
NCHUNK = E // CH
NP = 10240
RPT = NP // 16
f32 = jnp.float32
i32 = jnp.int32

_mesh = plsc.VectorSubcoreMesh(core_axis_name="c", subcore_axis_name="s")


_BN1 = 1000


def _proj_body(x_ref, w_ref, al_ref, ar_ref, ft_ref, el_ref, er_ref):
    y = jnp.dot(x_ref[...], w_ref[...], preferred_element_type=f32)
    for s in range(SLOTS):
        ft_ref[s] = y[:, s * 128:(s + 1) * 128]
    el_ref[...] = jnp.dot(y, al_ref[...], preferred_element_type=f32)
    er_ref[...] = jnp.dot(y, ar_ref[...], preferred_element_type=f32)


def _proj(nfeats, W_fc, AL, AR):
    return pl.pallas_call(
        _proj_body,
        grid=(N // _BN1,),
        in_specs=[
            pl.BlockSpec((_BN1, D_IN), lambda i: (i, 0)),
            pl.BlockSpec((D_IN, HEADS * D_OUT), lambda i: (0, 0)),
            pl.BlockSpec((HEADS * D_OUT, W), lambda i: (0, 0)),
            pl.BlockSpec((HEADS * D_OUT, W), lambda i: (0, 0)),
        ],
        out_specs=[
            pl.BlockSpec((SLOTS, _BN1, 128), lambda i: (0, i, 0)),
            pl.BlockSpec((_BN1, W), lambda i: (i, 0)),
            pl.BlockSpec((_BN1, W), lambda i: (i, 0)),
        ],
        out_shape=[
            jax.ShapeDtypeStruct((SLOTS, N, 128), f32),
            jax.ShapeDtypeStruct((N, W), f32),
            jax.ShapeDtypeStruct((N, W), f32),
        ],
    )(nfeats, W_fc, AL, AR)


_BE2 = 10000


def _ee_body(ef_ref, wfe_ref, ae_ref, hsel_ref, out_ref):
    ve = jnp.dot(wfe_ref[...] * ae_ref[...], hsel_ref[...],
                 preferred_element_type=f32)
    out_ref[...] = jnp.dot(ef_ref[...], ve, preferred_element_type=f32)


def _ee(efeats, W_fc_edge, ae_flat, HSEL):
    return pl.pallas_call(
        _ee_body,
        grid=(E // _BE2,),
        in_specs=[
            pl.BlockSpec((_BE2, D_EDGE), lambda i: (i, 0)),
            pl.BlockSpec((D_EDGE, HEADS * D_OUT), lambda i: (0, 0)),
            pl.BlockSpec((1, HEADS * D_OUT), lambda i: (0, 0)),
            pl.BlockSpec((HEADS * D_OUT, HP), lambda i: (0, 0)),
        ],
        out_specs=pl.BlockSpec((_BE2, HP), lambda i: (i, 0)),
        out_shape=jax.ShapeDtypeStruct((E, HP), f32),
    )(efeats, W_fc_edge, ae_flat, HSEL)


_P3 = 19


@functools.partial(
    pl.kernel,
    mesh=_mesh,
    out_type=jax.ShapeDtypeStruct((E, HP), f32),
    scratch_types=[
        pltpu.VMEM((CH,), i32), pltpu.VMEM((CH,), i32),
        pltpu.VMEM((CH, W), f32), pltpu.VMEM((CH, W), f32),
        pltpu.VMEM((CH, HP), f32),
        pltpu.VMEM((CH,), i32), pltpu.VMEM((CH,), i32),
        pltpu.VMEM((CH, W), f32), pltpu.VMEM((CH, W), f32),
        pltpu.VMEM((CH, HP), f32),
        pltpu.VMEM((CH, HP), f32),
        pltpu.SemaphoreType.DMA, pltpu.SemaphoreType.DMA,
        pltpu.SemaphoreType.DMA, pltpu.SemaphoreType.DMA,
        pltpu.SemaphoreType.DMA, pltpu.SemaphoreType.DMA,
    ],
)
def _edge_kernel(el_hbm, er_hbm, ee_hbm, src_hbm, dst_hbm, exr_out,
                 srcA, dstA, elA, erA, exA,
                 srcB, dstB, elB, erB, exB,
                 eev, mA, gA, oA, mB, gB, oB):
    cid = lax.axis_index("c")
    sid = lax.axis_index("s")
    wid = cid * 16 + sid
    A = (srcA, dstA, elA, erA, exA, mA, gA, oA)
    B = (srcB, dstB, elB, erB, exB, mB, gB, oB)
    bc = 39 * wid + jnp.minimum(wid, 2)

    def meta_issue(S, c):
        srcX, dstX, elX, erX, exX, mX, gX, oX = S
        base = c * CH
        pltpu.async_copy(src_hbm.at[pl.ds(base, CH)], srcX, mX)
        pltpu.async_copy(dst_hbm.at[pl.ds(base, CH)], dstX, mX)

    def meta_wait(S):
        srcX, dstX, elX, erX, exX, mX, gX, oX = S
        pltpu.make_async_copy(src_hbm.at[pl.ds(0, CH)], srcX, mX).wait()
        pltpu.make_async_copy(dst_hbm.at[pl.ds(0, CH)], dstX, mX).wait()

    def gather_issue(S):
        srcX, dstX, elX, erX, exX, mX, gX, oX = S
        pltpu.async_copy(el_hbm.at[srcX], elX, gX)
        pltpu.async_copy(er_hbm.at[dstX], erX, gX)

    def gather_wait(S):
        srcX, dstX, elX, erX, exX, mX, gX, oX = S
        pltpu.make_async_copy(el_hbm.at[srcX], elX, gX).wait()
        pltpu.make_async_copy(er_hbm.at[dstX], erX, gX).wait()

    def compute(S, c):
        srcX, dstX, elX, erX, exX, mX, gX, oX = S
        pltpu.sync_copy(ee_hbm.at[pl.ds(c * CH, CH), :], eev)

        def _row(j, cc):
            x = elX[j, pl.ds(0, 16)] + erX[j, pl.ds(0, 16)] + eev[j]
            x = jnp.maximum(x, x * 0.2)
            exX[j] = jnp.exp(x)
            return cc

        lax.fori_loop(0, CH, _row, 0)

    def out_issue(S, c):
        srcX, dstX, elX, erX, exX, mX, gX, oX = S
        pltpu.async_copy(exX, exr_out.at[pl.ds(c * CH, CH), :], oX)

    def out_wait(S):
        srcX, dstX, elX, erX, exX, mX, gX, oX = S
        pltpu.make_async_copy(exX, exr_out.at[pl.ds(0, CH), :], oX).wait()

    meta_issue(A, bc)
    meta_wait(A)
    gather_issue(A)
    meta_issue(B, bc + 1)

    def _body(p, carry):
        meta_wait(B)
        gather_issue(B)
        gather_wait(A)

        @pl.when(p > 0)
        def _():
            out_wait(A)

        compute(A, bc + 2 * p)
        out_issue(A, bc + 2 * p)

        @pl.when(p < _P3 - 1)
        def _():
            meta_issue(A, bc + 2 * p + 2)
            meta_wait(A)
            gather_issue(A)

        gather_wait(B)

        @pl.when(p > 0)
        def _():
            out_wait(B)

        compute(B, bc + 2 * p + 1)
        out_issue(B, bc + 2 * p + 1)

        @pl.when(p < _P3 - 1)
        def _():
            meta_issue(B, bc + 2 * p + 3)

        return carry

    lax.fori_loop(0, _P3, _body, 0)
    out_wait(A)
    out_wait(B)

    def _tail(c):
        meta_issue(A, c)
        meta_wait(A)
        gather_issue(A)
        gather_wait(A)
        compute(A, c)
        out_issue(A, c)
        out_wait(A)

    _tail(bc + 38)

    @pl.when(wid < 2)
    def _():
        _tail(bc + 39)


_BT = 16000


def _tr_body(x_ref, o_ref):
    o_ref[...] = x_ref[...].T[:HEADS, :]


def _transpose_ex(ex_rows):
    return pl.pallas_call(
        _tr_body,
        grid=(E // _BT,),
        in_specs=[pl.BlockSpec((_BT, HP), lambda i: (i, 0))],
        out_specs=pl.BlockSpec((HEADS, _BT), lambda i: (0, i)),
        out_shape=jax.ShapeDtypeStruct((HEADS, E), f32),
    )(ex_rows)


_CH4 = 64
_PAIRS = 78


@functools.partial(
    pl.kernel,
    mesh=_mesh,
    out_type=[
        jax.ShapeDtypeStruct((SLOTS, NP, 128), f32),
        jax.ShapeDtypeStruct((2, NP, W), f32),
    ],
    scratch_types=[
        pltpu.VMEM((_CH4,), i32), pltpu.VMEM((_CH4,), i32),
        pltpu.VMEM((_CH4,), i32), pltpu.VMEM((_CH4,), f32),
        pltpu.VMEM((_CH4, 128), f32), pltpu.VMEM((_CH4, HP), f32),
        pltpu.VMEM((_CH4,), i32), pltpu.VMEM((_CH4,), i32),
        pltpu.VMEM((_CH4,), i32), pltpu.VMEM((_CH4,), f32),
        pltpu.VMEM((_CH4, 128), f32), pltpu.VMEM((_CH4, HP), f32),
        pltpu.VMEM((64, 128), f32),
        pltpu.VMEM_SHARED((NP, 128), f32),
        pltpu.SemaphoreType.DMA, pltpu.SemaphoreType.DMA,
        pltpu.SemaphoreType.DMA, pltpu.SemaphoreType.DMA,
        pltpu.SemaphoreType.DMA, pltpu.SemaphoreType.DMA,
    ],
)
def _msg_kernel(ft_hbm, ext_hbm, exr_hbm, src_hbm, dst_hbm, rst_out, dn_out,
                srcA, dstA, idxA, wA, rowsA, exA,
                srcB, dstB, idxB, wB, rowsB, exB,
                zbuf, acc, mA, gA, sA, mB, gB, sB):
    cid = lax.axis_index("c")
    sid = lax.axis_index("s")
    A = (srcA, dstA, idxA, wA, rowsA, exA, mA, gA, sA)
    B = (srcB, dstB, idxB, wB, rowsB, exB, mB, gB, sB)

    def _zb(i, carry):
        for k in range(8):
            zbuf[i, pl.ds(k * 16, 16)] = jnp.zeros((16,), f32)
        return carry

    lax.fori_loop(0, 64, _zb, 0)

    bc = 156 * sid + jnp.minimum(sid, 4)

    def _zero_acc():
        for q in range(RPT // 64):
            pltpu.sync_copy(
                zbuf, acc.at[pl.ds(sid * RPT + q * 64, 64)])
        plsc.subcore_barrier()

    def meta_issue(S, c, h):
        srcX, dstX, idxX, wX, rowsX, exX, mX, gX, sX = S
        base = c * _CH4
        pltpu.async_copy(src_hbm.at[pl.ds(base, _CH4)], srcX, mX)
        pltpu.async_copy(dst_hbm.at[pl.ds(base, _CH4)], dstX, mX)
        pltpu.async_copy(ext_hbm.at[h, pl.ds(base, _CH4)], wX, mX)

    def meta_wait(S):
        srcX, dstX, idxX, wX, rowsX, exX, mX, gX, sX = S
        pltpu.make_async_copy(src_hbm.at[pl.ds(0, _CH4)], srcX, mX).wait()
        pltpu.make_async_copy(dst_hbm.at[pl.ds(0, _CH4)], dstX, mX).wait()
        pltpu.make_async_copy(ext_hbm.at[0, pl.ds(0, _CH4)], wX, mX).wait()

    def gather_issue(S, slot):
        srcX, dstX, idxX, wX, rowsX, exX, mX, gX, sX = S

        def _mkidx(g, c):
            idxX[pl.ds(g * 16, 16)] = srcX[pl.ds(g * 16, 16)] + slot * N
            return c

        lax.fori_loop(0, _CH4 // 16, _mkidx, 0)
        pltpu.async_copy(ft_hbm.at[idxX], rowsX, gX)

    def gather_wait(S):
        srcX, dstX, idxX, wX, rowsX, exX, mX, gX, sX = S
        pltpu.make_async_copy(ft_hbm.at[idxX], rowsX, gX).wait()

    def mult(S):
        srcX, dstX, idxX, wX, rowsX, exX, mX, gX, sX = S

        def _grp(g, c):
            wg = wX[pl.ds(g * 16, 16)]
            for l in range(16):
                ws = wg[l]
                j = g * 16 + l
                for k in range(8):
                    rowsX[j, pl.ds(k * 16, 16)] = (
                        rowsX[j, pl.ds(k * 16, 16)] * ws)
            return c

        lax.fori_loop(0, _CH4 // 16, _grp, 0)

    def scatter_issue(S):
        srcX, dstX, idxX, wX, rowsX, exX, mX, gX, sX = S
        pltpu.async_copy(rowsX, acc.at[dstX], sX, add=True)

    def scatter_wait(S):
        srcX, dstX, idxX, wX, rowsX, exX, mX, gX, sX = S
        pltpu.make_async_copy(ft_hbm.at[idxX], rowsX, sX).wait()

    def dn_meta_issue(S, c):
        srcX, dstX, idxX, wX, rowsX, exX, mX, gX, sX = S
        base = c * _CH4
        pltpu.async_copy(dst_hbm.at[pl.ds(base, _CH4)], dstX, mX)
        pltpu.async_copy(exr_hbm.at[pl.ds(base, _CH4), :], exX, mX)

    def dn_meta_wait(S):
        srcX, dstX, idxX, wX, rowsX, exX, mX, gX, sX = S
        pltpu.make_async_copy(dst_hbm.at[pl.ds(0, _CH4)], dstX, mX).wait()
        pltpu.make_async_copy(exr_hbm.at[pl.ds(0, _CH4), :], exX, mX).wait()

    def dn_fill(S):
        srcX, dstX, idxX, wX, rowsX, exX, mX, gX, sX = S

        def _drow(j, c):
            v = exX[j]
            for k in range(8):
                rowsX[j, pl.ds(k * 16, 16)] = v
            return c

        lax.fori_loop(0, _CH4, _drow, 0)

    _zero_acc()
    dn_meta_issue(A, bc)
    dn_meta_issue(B, bc + 1)

    def _dnbody(p, carry):
        dn_meta_wait(A)
        dn_fill(A)
        scatter_issue(A)
        dn_meta_wait(B)
        dn_fill(B)
        scatter_issue(B)

        @pl.when(p < _PAIRS - 1)
        def _():
            scatter_wait(A)
            dn_meta_issue(A, bc + 2 * p + 2)
            scatter_wait(B)
            dn_meta_issue(B, bc + 2 * p + 3)

        @pl.when(p == _PAIRS - 1)
        def _():
            scatter_wait(A)
            scatter_wait(B)

        return carry

    lax.fori_loop(0, _PAIRS, _dnbody, 0)

    @pl.when(sid < 4)
    def _():
        dn_meta_issue(A, bc + 156)
        dn_meta_wait(A)
        dn_fill(A)
        scatter_issue(A)
        scatter_wait(A)

    plsc.subcore_barrier()
    pltpu.sync_copy(acc.at[pl.ds(sid * RPT, RPT)],
                    dn_out.at[cid, pl.ds(sid * RPT, RPT)])
    plsc.subcore_barrier()

    def _head(h, hcarry):
        slot = h * 2 + cid
        _zero_acc()
        meta_issue(A, bc, h)
        meta_issue(B, bc + 1, h)
        meta_wait(A)
        gather_issue(A, slot)

        def _body(p, carry):
            meta_wait(B)
            gather_issue(B, slot)
            gather_wait(A)
            mult(A)
            scatter_issue(A)
            gather_wait(B)
            mult(B)
            scatter_issue(B)

            @pl.when(p < _PAIRS - 1)
            def _():
                scatter_wait(A)
                meta_issue(A, bc + 2 * p + 2, h)
                meta_wait(A)
                gather_issue(A, slot)
                scatter_wait(B)
                meta_issue(B, bc + 2 * p + 3, h)

            @pl.when(p == _PAIRS - 1)
            def _():
                scatter_wait(A)
                scatter_wait(B)

            return carry

        lax.fori_loop(0, _PAIRS, _body, 0)

        @pl.when(sid < 4)
        def _():
            meta_issue(A, bc + 156, h)
            meta_wait(A)
            gather_issue(A, slot)
            gather_wait(A)
            mult(A)
            scatter_issue(A)
            scatter_wait(A)

        plsc.subcore_barrier()
        pltpu.sync_copy(acc.at[pl.ds(sid * RPT, RPT)],
                        rst_out.at[slot, pl.ds(sid * RPT, RPT)])
        plsc.subcore_barrier()
        return hcarry

    lax.fori_loop(0, HEADS, _head, 0)


_BN5 = 1024


def _head_body(rst_ref, dn_ref, bias_ref, wps_ref, wpd_ref, s1_ref, s2_ref):
    dn = (dn_ref[0] + dn_ref[1]) * 0.5
    dn = jnp.where(dn == 0.0, 1.0, dn)
    h0 = jnp.zeros((_BN5, 128), f32)
    h1 = jnp.zeros((_BN5, 128), f32)
    for s in range(SLOTS):
        hh, cc = s // 2, s % 2
        t = rst_ref[s] / dn[:, hh:hh + 1] + bias_ref[s][None, :]
        t = jnp.maximum(t, 0.0)
        if cc == 0:
            h0 = h0 + t
        else:
            h1 = h1 + t
    h0 = h0 * 0.125
    h1 = h1 * 0.125
    s1_ref[...] = (jnp.dot(h0, wps_ref[0], preferred_element_type=f32)
                   + jnp.dot(h1, wps_ref[1], preferred_element_type=f32))
    s2_ref[...] = (jnp.dot(h0, wpd_ref[0], preferred_element_type=f32)
                   + jnp.dot(h1, wpd_ref[1], preferred_element_type=f32))


def _head_mean(rst, dn, bias16, WpS, WpD):
    return pl.pallas_call(
        _head_body,
        grid=(NP // _BN5,),
        in_specs=[
            pl.BlockSpec((SLOTS, _BN5, 128), lambda i: (0, i, 0)),
            pl.BlockSpec((2, _BN5, W), lambda i: (0, i, 0)),
            pl.BlockSpec((SLOTS, 128), lambda i: (0, 0)),
            pl.BlockSpec((2, 128, W), lambda i: (0, 0, 0)),
            pl.BlockSpec((2, 128, W), lambda i: (0, 0, 0)),
        ],
        out_specs=[
            pl.BlockSpec((_BN5, W), lambda i: (i, 0)),
            pl.BlockSpec((_BN5, W), lambda i: (i, 0)),
        ],
        out_shape=[
            jax.ShapeDtypeStruct((NP, W), f32),
            jax.ShapeDtypeStruct((NP, W), f32),
        ],
    )(rst, dn, bias16, WpS, WpD)


@functools.partial(
    pl.kernel,
    mesh=_mesh,
    out_type=jax.ShapeDtypeStruct((E, HP), f32),
    scratch_types=[
        pltpu.VMEM((CH,), i32), pltpu.VMEM((CH,), i32),
        pltpu.VMEM((CH, W), f32), pltpu.VMEM((CH, W), f32),
        pltpu.VMEM((CH, HP), f32),
        pltpu.VMEM((CH,), i32), pltpu.VMEM((CH,), i32),
        pltpu.VMEM((CH, W), f32), pltpu.VMEM((CH, W), f32),
        pltpu.VMEM((CH, HP), f32),
        pltpu.VMEM((W,), f32),
        pltpu.SemaphoreType.DMA, pltpu.SemaphoreType.DMA,
        pltpu.SemaphoreType.DMA, pltpu.SemaphoreType.DMA,
        pltpu.SemaphoreType.DMA, pltpu.SemaphoreType.DMA,
    ],
)
def _score_kernel(s1_hbm, s2_hbm, bp_hbm, src_hbm, dst_hbm, out_hbm,
                  srcA, dstA, avA, bvA, ovA,
                  srcB, dstB, avB, bvB, ovB,
                  bpv, mA, gA, oA, mB, gB, oB):
    cid = lax.axis_index("c")
    sid = lax.axis_index("s")
    wid = cid * 16 + sid
    A = (srcA, dstA, avA, bvA, ovA, mA, gA, oA)
    B = (srcB, dstB, avB, bvB, ovB, mB, gB, oB)
    bc = 39 * wid + jnp.minimum(wid, 2)
    pltpu.sync_copy(bp_hbm, bpv)

    def meta_issue(S, c):
        srcX, dstX, avX, bvX, ovX, mX, gX, oX = S
        base = c * CH
        pltpu.async_copy(src_hbm.at[pl.ds(base, CH)], srcX, mX)
        pltpu.async_copy(dst_hbm.at[pl.ds(base, CH)], dstX, mX)

    def meta_wait(S):
        srcX, dstX, avX, bvX, ovX, mX, gX, oX = S
        pltpu.make_async_copy(src_hbm.at[pl.ds(0, CH)], srcX, mX).wait()
        pltpu.make_async_copy(dst_hbm.at[pl.ds(0, CH)], dstX, mX).wait()

    def gather_issue(S):
        srcX, dstX, avX, bvX, ovX, mX, gX, oX = S
        pltpu.async_copy(s1_hbm.at[srcX], avX, gX)
        pltpu.async_copy(s2_hbm.at[dstX], bvX, gX)

    def gather_wait(S):
        srcX, dstX, avX, bvX, ovX, mX, gX, oX = S
        pltpu.make_async_copy(s1_hbm.at[srcX], avX, gX).wait()
        pltpu.make_async_copy(s2_hbm.at[dstX], bvX, gX).wait()

    def compute(S):
        srcX, dstX, avX, bvX, ovX, mX, gX, oX = S
        bb = bpv[pl.ds(0, 16)]

        def _row(j, c):
            ovX[j] = avX[j, pl.ds(0, 16)] + bvX[j, pl.ds(0, 16)] + bb
            return c

        lax.fori_loop(0, CH, _row, 0)

    def out_issue(S, c):
        srcX, dstX, avX, bvX, ovX, mX, gX, oX = S
        pltpu.async_copy(ovX, out_hbm.at[pl.ds(c * CH, CH), :], oX)

    def out_wait(S):
        srcX, dstX, avX, bvX, ovX, mX, gX, oX = S
        pltpu.make_async_copy(ovX, out_hbm.at[pl.ds(0, CH), :], oX).wait()

    meta_issue(A, bc)
    meta_wait(A)
    gather_issue(A)
    meta_issue(B, bc + 1)

    def _body(p, carry):
        meta_wait(B)
        gather_issue(B)
        gather_wait(A)

        @pl.when(p > 0)
        def _():
            out_wait(A)

        compute(A)
        out_issue(A, bc + 2 * p)

        @pl.when(p < _P3 - 1)
        def _():
            meta_issue(A, bc + 2 * p + 2)
            meta_wait(A)
            gather_issue(A)

        gather_wait(B)

        @pl.when(p > 0)
        def _():
            out_wait(B)

        compute(B)
        out_issue(B, bc + 2 * p + 1)

        @pl.when(p < _P3 - 1)
        def _():
            meta_issue(B, bc + 2 * p + 3)

        return carry

    lax.fori_loop(0, _P3, _body, 0)
    out_wait(A)
    out_wait(B)

    def _tail(c):
        meta_issue(A, c)
        meta_wait(A)
        gather_issue(A)
        gather_wait(A)
        compute(A)
        out_issue(A, c)
        out_wait(A)

    _tail(bc + 38)

    @pl.when(wid < 2)
    def _():
        _tail(bc + 39)


def kernel(nfeats, efeats, W_fc, attn_l, attn_r, W_fc_edge, attn_e, bias,
           W_pred, b_pred, edge_index):
    src = edge_index[0]
    dst = edge_index[1]

    colw = lax.broadcasted_iota(i32, (HEADS * D_OUT, W), 1)
    roww = lax.broadcasted_iota(i32, (HEADS * D_OUT, W), 0) // D_OUT
    selw = colw == roww
    AL = jnp.where(selw, attn_l.reshape(-1)[:, None], 0.0).astype(f32)
    AR = jnp.where(selw, attn_r.reshape(-1)[:, None], 0.0).astype(f32)
    col16 = lax.broadcasted_iota(i32, (HEADS * D_OUT, HP), 1)
    row16 = lax.broadcasted_iota(i32, (HEADS * D_OUT, HP), 0) // D_OUT
    HSEL = jnp.where(col16 == row16, 1.0, 0.0).astype(f32)
    bias16 = bias.reshape(SLOTS, 128)
    WpS = jnp.zeros((2, 128, W), f32).at[:, :, :2].set(
        W_pred[:D_OUT].reshape(2, 128, 2))
    WpD = jnp.zeros((2, 128, W), f32).at[:, :, :2].set(
        W_pred[D_OUT:].reshape(2, 128, 2))
    bp = jnp.zeros((W,), f32).at[:2].set(b_pred)

    ft, el, er = _proj(nfeats, W_fc, AL, AR)
    ee = _ee(efeats, W_fc_edge, attn_e.reshape(1, -1), HSEL)
    ex_rows = _edge_kernel(el, er, ee, src, dst)
    ext = _transpose_ex(ex_rows)
    rst, dn = _msg_kernel(ft.reshape(SLOTS * N, 128), ext, ex_rows, src, dst)
    s1, s2 = _head_mean(rst, dn, bias16, WpS, WpD)
    out16 = _score_kernel(s1, s2, bp, src, dst)
    return out16[:, :2]
